# baseline (speedup 1.0000x reference)
import functools

import jax
import jax.numpy as jnp
from jax import lax
from jax.experimental import pallas as pl
from jax.experimental.pallas import tpu as pltpu
from jax.experimental.pallas import tpu_sc as plsc

NC = 2
NS = 16
NW = NC * NS

BLK = 128
HSLOT = 2

TCOLS = 8192


def _relayout_kernel(in_ref, out_hbm, tr_v, sem):
    i = pl.program_id(0)
    tr_v[:, 0:32] = in_ref[...].T
    pltpu.async_copy(
        tr_v,
        out_hbm.at[pl.ds(i * TCOLS, TCOLS)],
        sem).wait()


def _gather_kernel(hist, table_hbm, idx_hbm, out_hbm,
                   idx_v, rows_a, rows_b, gsem_a, gsem_b, osem_a, osem_b):
    wid = lax.axis_index("s") * NC + lax.axis_index("c")
    b0 = wid * BLK

    pltpu.sync_copy(idx_hbm.at[:, pl.ds(b0, BLK)], idx_v)

    n_chunks = hist // HSLOT
    bufs = (rows_a, rows_b)
    gsems = (gsem_a, gsem_b)
    osems = (osem_a, osem_b)

    pend_out = [None, None]
    for c in range(n_chunks):
        p = c % 2
        buf, gsem, osem = bufs[p], gsems[p], osems[p]
        if pend_out[p] is not None:
            for cp in pend_out[p]:
                cp.wait()
        gathers = []
        for s in range(HSLOT):
            h = c * HSLOT + s
            gathers.append(pltpu.async_copy(
                table_hbm.at[idx_v.at[h]],
                buf.at[s],
                gsem))
        writes = []
        for s in range(HSLOT):
            h = c * HSLOT + s
            gathers[s].wait()
            writes.append(pltpu.async_copy(
                buf.at[s, :, pl.ds(0, 32)],
                out_hbm.at[pl.ds(b0, BLK), h],
                osem))
        pend_out[p] = writes
    for writes in pend_out:
        if writes is not None:
            for cp in writes:
                cp.wait()


def kernel(inputs, embeddings):
    batch, hist = inputs.shape
    in_dim, out_dim = embeddings.shape
    assert batch == NW * BLK
    assert hist % HSLOT == 0
    assert (TCOLS * out_dim) % 128 == 0

    emb_t = embeddings.T
    n_blocks = -(-in_dim // TCOLS)
    table = pl.pallas_call(
        _relayout_kernel,
        grid=(n_blocks,),
        in_specs=[pl.BlockSpec((out_dim, TCOLS), lambda i: (0, i))],
        out_specs=pl.BlockSpec(memory_space=pl.ANY),
        out_shape=jax.ShapeDtypeStruct((in_dim, 128), jnp.float32),
        scratch_shapes=[pltpu.VMEM((TCOLS, 128), jnp.float32),
                        pltpu.SemaphoreType.DMA],
    )(emb_t)

    idx_t = inputs.T

    mesh = plsc.VectorSubcoreMesh(
        core_axis_name="c", subcore_axis_name="s",
        num_cores=NC, num_subcores=NS)

    grab = pl.kernel(
        functools.partial(_gather_kernel, hist),
        out_type=jax.ShapeDtypeStruct((batch, hist, out_dim), jnp.float32),
        mesh=mesh,
        scratch_types=[
            pltpu.VMEM((hist, BLK), jnp.int32),
            pltpu.VMEM((HSLOT, BLK, 128), jnp.float32),
            pltpu.VMEM((HSLOT, BLK, 128), jnp.float32),
            pltpu.SemaphoreType.DMA,
            pltpu.SemaphoreType.DMA,
            pltpu.SemaphoreType.DMA,
            pltpu.SemaphoreType.DMA,
        ],
        compiler_params=pltpu.CompilerParams(use_tc_tiling_on_sc=False),
    )
    return grab(table, idx_t)


# speedup vs baseline: 1.4680x; 1.4680x over previous
import functools

import jax
import jax.numpy as jnp
from jax import lax
from jax.experimental import pallas as pl
from jax.experimental.pallas import tpu as pltpu
from jax.experimental.pallas import tpu_sc as plsc

NC = 2
NS = 16
NW = NC * NS

BLK = 128
HSLOT = 5

TCOLS = 8192


def _relayout_kernel(in_ref, out_ref):
    q = TCOLS // 4
    for m in range(4):
        out_ref[:, m * 32:(m + 1) * 32] = in_ref[:, m * q:(m + 1) * q].T


def _gather_kernel(hist, table_hbm, idx_hbm, out_hbm,
                   idx_v, rows_a, rows_b, gsem_a, gsem_b, osem_a, osem_b):
    wid = lax.axis_index("s") * NC + lax.axis_index("c")
    b0 = wid * BLK

    pltpu.sync_copy(idx_hbm.at[:, pl.ds(b0, BLK)], idx_v)

    n_chunks = hist // HSLOT
    bufs = (rows_a, rows_b)
    gsems = (gsem_a, gsem_b)
    osems = (osem_a, osem_b)

    pend_out = [None, None]
    for c in range(n_chunks):
        p = c % 2
        buf, gsem, osem = bufs[p], gsems[p], osems[p]
        if pend_out[p] is not None:
            for cp in pend_out[p]:
                cp.wait()
        gathers = []
        for s in range(HSLOT):
            h = c * HSLOT + s
            gathers.append(pltpu.async_copy(
                table_hbm.at[idx_v.at[h]],
                buf.at[s],
                gsem))
        writes = []
        for s in range(HSLOT):
            h = c * HSLOT + s
            gathers[s].wait()
            writes.append(pltpu.async_copy(
                buf.at[s],
                out_hbm.at[pl.ds(b0, BLK), h],
                osem))
        pend_out[p] = writes
    for writes in pend_out:
        if writes is not None:
            for cp in writes:
                cp.wait()


def kernel(inputs, embeddings):
    batch, hist = inputs.shape
    in_dim, out_dim = embeddings.shape
    assert batch == NW * BLK
    assert hist % HSLOT == 0
    assert (TCOLS * out_dim) % 128 == 0

    emb_t = embeddings.T
    n_blocks = -(-in_dim // TCOLS)
    lin = pl.pallas_call(
        _relayout_kernel,
        grid=(n_blocks,),
        in_specs=[pl.BlockSpec((out_dim, TCOLS), lambda i: (0, i))],
        out_specs=pl.BlockSpec((TCOLS // 4, 128), lambda i: (i, 0)),
        out_shape=jax.ShapeDtypeStruct((n_blocks * (TCOLS // 4), 128),
                                       jnp.float32),
    )(emb_t)
    table = lin.reshape(n_blocks * TCOLS, out_dim)

    sig = ((inputs & ~jnp.int32(8191))
           | ((inputs & jnp.int32(2047)) << 2)
           | ((inputs >> 11) & jnp.int32(3)))
    idx_t = sig.T

    mesh = plsc.VectorSubcoreMesh(
        core_axis_name="c", subcore_axis_name="s",
        num_cores=NC, num_subcores=NS)

    grab = pl.kernel(
        functools.partial(_gather_kernel, hist),
        out_type=jax.ShapeDtypeStruct((batch, hist, out_dim), jnp.float32),
        mesh=mesh,
        scratch_types=[
            pltpu.VMEM((hist, BLK), jnp.int32),
            pltpu.VMEM((HSLOT, BLK, out_dim), jnp.float32),
            pltpu.VMEM((HSLOT, BLK, out_dim), jnp.float32),
            pltpu.SemaphoreType.DMA,
            pltpu.SemaphoreType.DMA,
            pltpu.SemaphoreType.DMA,
            pltpu.SemaphoreType.DMA,
        ],
        compiler_params=pltpu.CompilerParams(use_tc_tiling_on_sc=False),
    )
    return grab(table, idx_t)
